# baseline (speedup 1.0000x reference)
import jax
import jax.numpy as jnp
from jax import lax
from jax.experimental import pallas as pl
from jax.experimental.pallas import tpu as pltpu
from jax.experimental.pallas import tpu_sc as plsc

_BINS = (1, 2, 3, 4, 5, 6, 7, 8, 12, 16, 20, 24, 32, 64, 128)
_N = 1048576
_D = 20
_DP = 24
_CT = _DP // 8
_NB = _N // 128
_NC = 2
_NS = 16
_L = 16
_NW = _NC * _NS
_BW = _N // _NW
_NBCH = 8
_CHUNK = _NBCH * 128
_NCHUNK = _BW // _CHUNK


def _width_body(lengths_hbm, tablef_hbm, out_hbm,
                table_f, lut_v, len_v, stage_v, sem0, sem1):
    wid = lax.axis_index("s") * _NC + lax.axis_index("c")
    base_w = wid * _BW
    base_nb = wid * (_BW // 128)

    pltpu.sync_copy(tablef_hbm, table_f)
    pltpu.sync_copy(lengths_hbm.at[pl.ds(base_w, _BW)], len_v)

    iota = lax.iota(jnp.int32, _L)
    zero_v = jnp.zeros((_L,), jnp.int32)
    one_v = jnp.full((_L,), 1, jnp.int32)
    cap_v = jnp.full((_L,), 255, jnp.int32)

    for k in range(256 // _L):
        vals = iota + jnp.full((_L,), k * _L, jnp.int32)
        cnt = jnp.zeros((_L,), jnp.int32)
        for b in _BINS:
            d = vals - jnp.full((_L,), b, jnp.int32)
            cnt = cnt + jnp.minimum(jnp.maximum(d, zero_v), one_v)
        lut_v[pl.ds(k * _L, _L)] = cnt

    pitch_v = jnp.full((_L,), _DP, jnp.int32)
    sems = (sem0, sem1)

    def compute_chunk(ci, b):
        buf = stage_v.at[b]

        def load_group(vo, vi):
            lv = len_v[pl.ds(ci * _CHUNK + vo * 128 + vi * _L, _L)]
            cl = jnp.minimum(jnp.maximum(lv, zero_v), cap_v)
            idx = plsc.load_gather(lut_v, [cl])
            src = idx * pitch_v
            vals = []
            for _c in range(_D):
                vals.append(plsc.load_gather(table_f, [src]))
                src = src + one_v
            return vals

        def store_group(vo, vi, vals):
            for c in range(_D):
                buf[c // 8, vo, c % 8, pl.ds(vi * _L, _L)] = vals[c]

        def nb_body(vo, _):
            vals = load_group(vo, 0)
            for vi in range(1, 8):
                nxt = load_group(vo, vi)
                store_group(vo, vi - 1, vals)
                vals = nxt
            store_group(vo, 7, vals)
            return ()

        lax.fori_loop(0, _NBCH, nb_body, ())

    def start_out(ci, b):
        nb0 = base_nb + ci * _NBCH
        for ct in range(_CT):
            pltpu.async_copy(stage_v.at[b, ct],
                             out_hbm.at[ct, pl.ds(nb0, _NBCH)], sems[b])

    def wait_out(ci, b):
        nb0 = base_nb + ci * _NBCH
        for ct in range(_CT):
            pltpu.make_async_copy(stage_v.at[b, ct],
                                  out_hbm.at[ct, pl.ds(nb0, _NBCH)],
                                  sems[b]).wait()

    _SKIP_COMPUTE = False

    def compute_chunk_p(ci, b):
        if not _SKIP_COMPUTE:
            compute_chunk(ci, b)

    for b in range(2):
        compute_chunk_p(b, b)
        start_out(b, b)

    def pair_body(g, _):
        for b in range(2):
            ci = 2 + 2 * g + b
            wait_out(ci, b)
            compute_chunk_p(ci, b)
            start_out(ci, b)
        return ()

    lax.fori_loop(0, (_NCHUNK - 2) // 2, pair_body, ())

    for b in range(2):
        wait_out(0, b)


@jax.jit
def _width(lengths, table):
    table_f = jnp.pad(table, ((0, 0), (0, _DP - _D))).reshape(16 * _DP)
    mesh = plsc.VectorSubcoreMesh(
        core_axis_name="c", subcore_axis_name="s",
        num_cores=_NC, num_subcores=_NS,
    )
    out4 = pl.kernel(
        _width_body,
        out_type=jax.ShapeDtypeStruct((_CT, _NB, 8, 128), jnp.float32),
        mesh=mesh,
        compiler_params=pltpu.CompilerParams(
            needs_layout_passes=False,
            use_tc_tiling_on_sc=False,
        ),
        scratch_types=[
            pltpu.VMEM((16 * _DP,), jnp.float32),
            pltpu.VMEM((256,), jnp.int32),
            pltpu.VMEM((_BW,), jnp.int32),
            pltpu.VMEM((2, _CT, _NBCH, 8, 128), jnp.float32),
            pltpu.SemaphoreType.DMA,
            pltpu.SemaphoreType.DMA,
        ],
    )(lengths, table_f)
    return out4.transpose((1, 3, 0, 2)).reshape(_N, _DP)[:, :_D]


def kernel(lengths, table):
    return _width(lengths, table)


# speedup vs baseline: 1.2255x; 1.2255x over previous
import jax
import jax.numpy as jnp
from jax import lax
from jax.experimental import pallas as pl
from jax.experimental.pallas import tpu as pltpu
from jax.experimental.pallas import tpu_sc as plsc

_BINS = (1, 2, 3, 4, 5, 6, 7, 8, 12, 16, 20, 24, 32, 64, 128)
_N = 1048576
_D = 20
_DP = 24
_TP = 25
_CT = _DP // 8
_NB = _N // 128
_NC = 2
_NS = 16
_L = 16
_NW = _NC * _NS
_BW = _N // _NW
_NBCH = 8
_CHUNK = _NBCH * 128
_NCHUNK = _BW // _CHUNK


def _width_body(lengths_hbm, tablef_hbm, out_hbm,
                table_f, lut_v, len_v, stage_v, sem0, sem1):
    wid = lax.axis_index("s") * _NC + lax.axis_index("c")
    base_w = wid * _BW
    base_nb = wid * (_BW // 128)

    pltpu.sync_copy(tablef_hbm, table_f)
    pltpu.sync_copy(lengths_hbm.at[pl.ds(base_w, _BW)], len_v)

    iota = lax.iota(jnp.int32, _L)
    zero_v = jnp.zeros((_L,), jnp.int32)
    one_v = jnp.full((_L,), 1, jnp.int32)
    cap_v = jnp.full((_L,), 255, jnp.int32)

    for k in range(256 // _L):
        vals = iota + jnp.full((_L,), k * _L, jnp.int32)
        cnt = jnp.zeros((_L,), jnp.int32)
        for b in _BINS:
            d = vals - jnp.full((_L,), b, jnp.int32)
            cnt = cnt + jnp.minimum(jnp.maximum(d, zero_v), one_v)
        lut_v[pl.ds(k * _L, _L)] = cnt

    pitch_v = jnp.full((_L,), _TP, jnp.int32)
    sems = (sem0, sem1)

    def compute_chunk(ci, b):
        buf = stage_v.at[b]

        def load_group(vo, vi):
            lv = len_v[pl.ds(ci * _CHUNK + vo * 128 + vi * _L, _L)]
            cl = jnp.minimum(jnp.maximum(lv, zero_v), cap_v)
            idx = plsc.load_gather(lut_v, [cl])
            src = idx * pitch_v
            vals = []
            for _c in range(_D):
                vals.append(plsc.load_gather(table_f, [src]))
                src = src + one_v
            return vals

        def store_group(vo, vi, vals):
            for c in range(_D):
                buf[c // 8, vo, c % 8, pl.ds(vi * _L, _L)] = vals[c]

        def nb_body(vo, _):
            vals = load_group(vo, 0)
            for vi in range(1, 8):
                nxt = load_group(vo, vi)
                store_group(vo, vi - 1, vals)
                vals = nxt
            store_group(vo, 7, vals)
            return ()

        lax.fori_loop(0, _NBCH, nb_body, ())

    def start_out(ci, b):
        nb0 = base_nb + ci * _NBCH
        for ct in range(_CT):
            pltpu.async_copy(stage_v.at[b, ct],
                             out_hbm.at[ct, pl.ds(nb0, _NBCH)], sems[b])

    def wait_out(ci, b):
        nb0 = base_nb + ci * _NBCH
        for ct in range(_CT):
            pltpu.make_async_copy(stage_v.at[b, ct],
                                  out_hbm.at[ct, pl.ds(nb0, _NBCH)],
                                  sems[b]).wait()

    _SKIP_COMPUTE = False

    def compute_chunk_p(ci, b):
        if not _SKIP_COMPUTE:
            compute_chunk(ci, b)

    for b in range(2):
        compute_chunk_p(b, b)
        start_out(b, b)

    def pair_body(g, _):
        for b in range(2):
            ci = 2 + 2 * g + b
            wait_out(ci, b)
            compute_chunk_p(ci, b)
            start_out(ci, b)
        return ()

    lax.fori_loop(0, (_NCHUNK - 2) // 2, pair_body, ())

    for b in range(2):
        wait_out(0, b)


@jax.jit
def _width(lengths, table):
    table_f = jnp.pad(table, ((0, 0), (0, _TP - _D))).reshape(16 * _TP)
    mesh = plsc.VectorSubcoreMesh(
        core_axis_name="c", subcore_axis_name="s",
        num_cores=_NC, num_subcores=_NS,
    )
    out4 = pl.kernel(
        _width_body,
        out_type=jax.ShapeDtypeStruct((_CT, _NB, 8, 128), jnp.float32),
        mesh=mesh,
        compiler_params=pltpu.CompilerParams(
            needs_layout_passes=False,
            use_tc_tiling_on_sc=False,
        ),
        scratch_types=[
            pltpu.VMEM((16 * _TP,), jnp.float32),
            pltpu.VMEM((256,), jnp.int32),
            pltpu.VMEM((_BW,), jnp.int32),
            pltpu.VMEM((2, _CT, _NBCH, 8, 128), jnp.float32),
            pltpu.SemaphoreType.DMA,
            pltpu.SemaphoreType.DMA,
        ],
    )(lengths, table_f)
    return out4.transpose((1, 3, 0, 2)).reshape(_N, _DP)[:, :_D]


def kernel(lengths, table):
    return _width(lengths, table)
